# Initial kernel scaffold; baseline (speedup 1.0000x reference)
import jax
import jax.numpy as jnp
from jax.experimental import pallas as pl


def kernel(x, adj, W1, b1, W2, b2, W3, b3, W4, b4, W5, b5):
    raise NotImplementedError("write your pallas kernel here")


# speedup vs baseline: 1.0001x; 1.0001x over previous
import jax
import jax.numpy as jnp
from jax.experimental import pallas as pl
from jax.experimental.pallas import tpu as pltpu

BM = 400


def _pass1_body(adj_ref, x_ref, w1_ref, b1_ref, w2_ref, out_ref):
    acc = jnp.dot(adj_ref[...], x_ref[...], preferred_element_type=jnp.float32)
    h = jnp.dot(acc, w1_ref[...], preferred_element_type=jnp.float32)
    h = jnp.maximum(h + b1_ref[...], 0.0)
    out_ref[...] = jnp.dot(h, w2_ref[...], preferred_element_type=jnp.float32)


def _pass2_body(adj_ref, s2_ref, b2_ref, out_ref):
    t = jnp.dot(adj_ref[...], s2_ref[...], preferred_element_type=jnp.float32)
    out_ref[...] = jnp.max(t + b2_ref[...], axis=0, keepdims=True)[None]


def _head_body(pm_ref, w3_ref, b3_ref, w4_ref, b4_ref, w5_ref, b5_ref, out_ref):
    v = jnp.max(pm_ref[...], axis=(0, 1), keepdims=False)[None]
    v = jnp.maximum(jnp.dot(v, w3_ref[...], preferred_element_type=jnp.float32)
                    + b3_ref[...], 0.0)
    v = jnp.maximum(jnp.dot(v, w4_ref[...], preferred_element_type=jnp.float32)
                    + b4_ref[...], 0.0)
    out_ref[...] = (jnp.dot(v, w5_ref[...], preferred_element_type=jnp.float32)
                    + b5_ref[...])


def kernel(x, adj, W1, b1, W2, b2, W3, b3, W4, b4, W5, b5):
    n, nfeat = x.shape
    nhid = W1.shape[1]
    n2 = W2.shape[1]
    ncls = W5.shape[1]
    grid = (n // BM,)

    s2 = pl.pallas_call(
        _pass1_body,
        grid=grid,
        in_specs=[
            pl.BlockSpec((BM, n), lambda m: (m, 0)),
            pl.BlockSpec((n, nfeat), lambda m: (0, 0)),
            pl.BlockSpec((nfeat, nhid), lambda m: (0, 0)),
            pl.BlockSpec((1, nhid), lambda m: (0, 0)),
            pl.BlockSpec((nhid, n2), lambda m: (0, 0)),
        ],
        out_specs=pl.BlockSpec((BM, n2), lambda m: (m, 0)),
        out_shape=jax.ShapeDtypeStruct((n, n2), jnp.float32),
        compiler_params=pltpu.CompilerParams(
            dimension_semantics=("parallel",)),
    )(adj, x, W1, b1.reshape(1, -1), W2)

    part_max = pl.pallas_call(
        _pass2_body,
        grid=grid,
        in_specs=[
            pl.BlockSpec((BM, n), lambda m: (m, 0)),
            pl.BlockSpec((n, n2), lambda m: (0, 0)),
            pl.BlockSpec((1, n2), lambda m: (0, 0)),
        ],
        out_specs=pl.BlockSpec((1, 1, n2), lambda m: (m, 0, 0)),
        out_shape=jax.ShapeDtypeStruct((n // BM, 1, n2), jnp.float32),
        compiler_params=pltpu.CompilerParams(
            dimension_semantics=("parallel",)),
    )(adj, s2, b2.reshape(1, -1))

    out = pl.pallas_call(
        _head_body,
        in_specs=[
            pl.BlockSpec(part_max.shape, lambda: (0, 0, 0)),
            pl.BlockSpec(W3.shape, lambda: (0, 0)),
            pl.BlockSpec((1, W3.shape[1]), lambda: (0, 0)),
            pl.BlockSpec(W4.shape, lambda: (0, 0)),
            pl.BlockSpec((1, W4.shape[1]), lambda: (0, 0)),
            pl.BlockSpec(W5.shape, lambda: (0, 0)),
            pl.BlockSpec((1, ncls), lambda: (0, 0)),
        ],
        out_specs=pl.BlockSpec((1, ncls), lambda: (0, 0)),
        out_shape=jax.ShapeDtypeStruct((1, ncls), jnp.float32),
    )(part_max, W3, b3.reshape(1, -1), W4, b4.reshape(1, -1),
      W5, b5.reshape(1, -1))

    return out.reshape(ncls)
